# Initial kernel scaffold; baseline (speedup 1.0000x reference)
import jax
import jax.numpy as jnp
from jax.experimental import pallas as pl


def kernel(pos_query, pos_compose, node_attr_compose_sca, node_attr_compose_vec, params, edge_index_q_cps_knn):
    raise NotImplementedError("write your pallas kernel here")


# speedup vs baseline: 16.8437x; 16.8437x over previous
import functools

import jax
import jax.numpy as jnp
import numpy as np
from jax import lax
from jax.experimental import pallas as pl
from jax.experimental.pallas import tpu as pltpu
from jax.experimental.pallas import tpu_sc as plsc

_EPS = 1e-6
_CUT = 10.0
_NQ, _NCTX, _E = 15000, 50000, 480000
_EC = 16

_NC, _NS = 2, 16
_NW = _NC * _NS
_EPW = _E // _NW
_CH = 120
_NCHUNK = _EPW // _CH
_NQP = 15360
_RPT = _NQP // _NS


def _kron3(W):
    return jnp.kron(W.T, jnp.eye(3, dtype=W.dtype))


def _red3(n):
    return np.kron(np.eye(n, dtype=np.float32), np.ones((3, 1), np.float32))


def _exp3(n):
    return np.kron(np.eye(n, dtype=np.float32), np.ones((1, 3), np.float32))


def _node_body(sca, nvf, pc8, A1, R3, Wsn, Wss, A2, Gw, gb, E3, out):
    inter = jnp.dot(nvf[...], A1[...], preferred_element_type=jnp.float32)
    nrm = jnp.sqrt(jnp.dot(inter * inter, R3[...],
                           preferred_element_type=jnp.float32) + 1e-12)
    s = (jnp.dot(nrm, Wsn[...], preferred_element_type=jnp.float32)
         + jnp.dot(sca[...], Wss[...], preferred_element_type=jnp.float32))
    vv = jnp.dot(inter, A2[...], preferred_element_type=jnp.float32)
    gate = jax.nn.sigmoid(
        jnp.dot(s, Gw[...], preferred_element_type=jnp.float32) + gb[...])
    vv = vv * jnp.dot(gate, E3[...], preferred_element_type=jnp.float32)
    pad = jnp.zeros((s.shape[0], 8), jnp.float32)
    out[...] = jnp.concatenate([s, vv, pc8[...], pad], axis=1)


def _tc_node(sca, nvf, pc8, ws):
    bn = 2000
    grid = (_NCTX // bn,)
    row = lambda shape: pl.BlockSpec(shape, lambda i: (i, 0))
    full = lambda a: pl.BlockSpec(a.shape, lambda i: (0,) * a.ndim)
    return pl.pallas_call(
        _node_body,
        grid=grid,
        in_specs=[row((bn, 128)), row((bn, 96)), row((bn, 8))] + [full(a) for a in ws],
        out_specs=row((bn, 128)),
        out_shape=jax.ShapeDtypeStruct((_NCTX, 128), jnp.float32),
    )(sca, nvf, pc8, *ws)


def _sc_gather(T, pq8, dst, src):
    mesh = plsc.VectorSubcoreMesh(core_axis_name="c", subcore_axis_name="s",
                                  num_cores=_NC, num_subcores=_NS)

    @functools.partial(
        pl.kernel,
        out_type=[jax.ShapeDtypeStruct((_E, 128), jnp.float32),
                  jax.ShapeDtypeStruct((_E, 8), jnp.float32)],
        mesh=mesh,
        scratch_types=[pltpu.VMEM((_CH,), jnp.int32),
                       pltpu.VMEM((_CH,), jnp.int32),
                       pltpu.VMEM((_CH, 128), jnp.float32),
                       pltpu.VMEM((_CH, 8), jnp.float32),
                       pltpu.SemaphoreType.DMA,
                       pltpu.SemaphoreType.DMA],
        compiler_params=pltpu.CompilerParams(use_tc_tiling_on_sc=False),
    )
    def gather_k(t_hbm, pq_hbm, dst_hbm, src_hbm, g_hbm, gq_hbm,
                 idx_d, idx_s, rows_t, rows_q, sem1, sem2):
        w = lax.axis_index("s") * _NC + lax.axis_index("c")
        base0 = w * _EPW

        def body(j, carry):
            base = base0 + j * _CH
            pltpu.sync_copy(dst_hbm.at[pl.ds(base, _CH)], idx_d)
            pltpu.sync_copy(src_hbm.at[pl.ds(base, _CH)], idx_s)
            c1 = pltpu.async_copy(t_hbm.at[idx_d], rows_t, sem1)
            c2 = pltpu.async_copy(pq_hbm.at[idx_s], rows_q, sem2)
            c1.wait()
            c2.wait()
            pltpu.sync_copy(rows_t, g_hbm.at[pl.ds(base, _CH)])
            pltpu.sync_copy(rows_q, gq_hbm.at[pl.ds(base, _CH)])
            return carry

        lax.fori_loop(0, _NCHUNK, body, 0)

    return gather_k(T, pq8, dst, src)


def _edge_body(g, gq, offs, Kevf, A1e, R3e, Wsne, Wsse, A2e, Gwe, gbe, E3e,
               Aact, scaW, scab, e2nW, e2nb, n2eW, n2eb, Avn, bvn,
               A1o, R3o, Wsno, Wsso, A2o, Gwo, gbo, E3o, os_out, ov_out):
    dot = lambda a, b: jnp.dot(a, b, preferred_element_type=jnp.float32)
    ns_e = g[:, 0:64]
    nv_e = g[:, 64:112]
    pc = g[:, 112:120]
    vec = gq[...] - pc
    d2 = jnp.sum(vec * vec, axis=1, keepdims=True)
    dist = jnp.sqrt(d2 + 1e-12)
    step = _CUT / (_EC - 1)
    coeff = -0.5 / step ** 2
    esf = jnp.exp(coeff * (dist - offs[...]) ** 2)
    vnorm = vec / (dist + 1e-7)
    evf = dot(vnorm, Kevf[...])
    inter = dot(evf, A1e[...])
    nrm = jnp.sqrt(dot(inter * inter, R3e[...]) + 1e-12)
    s = dot(nrm, Wsne[...]) + dot(esf, Wsse[...])
    vv = dot(inter, A2e[...])
    gate = jax.nn.sigmoid(dot(s, Gwe[...]) + gbe[...])
    vv = vv * dot(gate, E3e[...])
    es = jnp.where(s >= 0, s, 0.01 * s)
    dvec = dot(vv, Aact[...])
    vdot = dot(vv * dvec, R3e[...])
    dns = dot(dvec * dvec, R3e[...])
    mask = (vdot >= 0.).astype(jnp.float32)
    corr = vv - dot(vdot / (dns + _EPS), E3e[...]) * dvec
    ev = 0.2 * vv + 0.8 * (dot(mask, E3e[...]) * vv
                           + dot(1. - mask, E3e[...]) * corr)
    y_s = ns_e * (dot(es, scaW[...]) + scab[...])
    y_v = dot(dot(es, e2nW[...]) + e2nb[...], E3o[...]) * nv_e
    y_v = y_v + (dot(dot(ns_e, n2eW[...]) + n2eb[...], E3o[...])
                 * (dot(ev, Avn[...]) + bvn[...]))
    inter2 = dot(y_v, A1o[...])
    nrm2 = jnp.sqrt(dot(inter2 * inter2, R3o[...]) + 1e-12)
    o_s = dot(nrm2, Wsno[...]) + dot(y_s, Wsso[...])
    o_v = dot(inter2, A2o[...])
    gate2 = jax.nn.sigmoid(dot(o_s, Gwo[...]) + gbo[...])
    o_v = o_v * dot(gate2, E3o[...])
    C = 0.5 * (jnp.cos(dist * (np.pi / _CUT)) + 1.0)
    C = C * (dist <= _CUT).astype(jnp.float32)
    os_out[...] = o_s * C
    ov_out[...] = o_v * C


def _tc_edge(G, Gq, ws):
    bn = 2000
    grid = (_E // bn,)
    row = lambda shape: pl.BlockSpec(shape, lambda i: (i, 0))
    full = lambda a: pl.BlockSpec(a.shape, lambda i: (0,) * a.ndim)
    return pl.pallas_call(
        _edge_body,
        grid=grid,
        in_specs=[row((bn, 128)), row((bn, 8))] + [full(a) for a in ws],
        out_specs=[row((bn, 64)), row((bn, 48))],
        out_shape=[jax.ShapeDtypeStruct((_E, 64), jnp.float32),
                   jax.ShapeDtypeStruct((_E, 48), jnp.float32)],
    )(G, Gq, *ws)


def _sc_scatter_one(O, src, z, width):
    mesh = plsc.VectorSubcoreMesh(core_axis_name="c", subcore_axis_name="s",
                                  num_cores=_NC, num_subcores=_NS)

    @functools.partial(
        pl.kernel,
        out_type=jax.ShapeDtypeStruct((_NC, _NQP, width), jnp.float32),
        mesh=mesh,
        scratch_types=[pltpu.VMEM((_CH,), jnp.int32),
                       pltpu.VMEM((_CH, width), jnp.float32),
                       pltpu.VMEM_SHARED((_NQP, width), jnp.float32)],
        compiler_params=pltpu.CompilerParams(use_tc_tiling_on_sc=False),
    )
    def scatter_k(o_hbm, src_hbm, z_hbm, p_hbm, idx_v, buf, acc):
        c = lax.axis_index("c")
        s = lax.axis_index("s")
        w = s * _NC + c
        t0 = s * _RPT
        pltpu.sync_copy(z_hbm, acc.at[pl.ds(t0, _RPT)])
        plsc.subcore_barrier()

        def body(j, carry):
            base = w * _EPW + j * _CH
            pltpu.sync_copy(src_hbm.at[pl.ds(base, _CH)], idx_v)
            pltpu.sync_copy(o_hbm.at[pl.ds(base, _CH)], buf)
            pltpu.sync_copy(buf, acc.at[idx_v], add=True)
            return carry

        lax.fori_loop(0, _NCHUNK, body, 0)
        plsc.subcore_barrier()
        pltpu.sync_copy(acc.at[pl.ds(t0, _RPT)], p_hbm.at[c, pl.ds(t0, _RPT)])

    return scatter_k(O, src, z)


def _sc_scatter(Os, Ov, src, zs, zv):
    return (_sc_scatter_one(Os, src, zs, 64),
            _sc_scatter_one(Ov, src, zv, 48))


def _final_body(ps, pv, A1c, R3c, Wsnc, Wssc, A2c, Gwc, gbc, E3c, Aactc,
                A1g, R3g, Wsng, Wssg, out):
    dot = lambda a, b: jnp.dot(a, b, preferred_element_type=jnp.float32)
    agg_s = ps[0] + ps[1]
    agg_v = pv[0] + pv[1]
    inter = dot(agg_v, A1c[...])
    nrm = jnp.sqrt(dot(inter * inter, R3c[...]) + 1e-12)
    s = dot(nrm, Wsnc[...]) + dot(agg_s, Wssc[...])
    vv = dot(inter, A2c[...])
    gate = jax.nn.sigmoid(dot(s, Gwc[...]) + gbc[...])
    vv = vv * dot(gate, E3c[...])
    cs = jnp.where(s >= 0, s, 0.01 * s)
    dvec = dot(vv, Aactc[...])
    vdot = dot(vv * dvec, R3c[...])
    dns = dot(dvec * dvec, R3c[...])
    mask = (vdot >= 0.).astype(jnp.float32)
    corr = vv - dot(vdot / (dns + _EPS), E3c[...]) * dvec
    cv = 0.2 * vv + 0.8 * (dot(mask, E3c[...]) * vv
                           + dot(1. - mask, E3c[...]) * corr)
    inter2 = dot(cv, A1g[...])
    nrm2 = jnp.sqrt(dot(inter2 * inter2, R3g[...]) + 1e-12)
    out[...] = dot(nrm2, Wsng[...]) + dot(cs, Wssg[...])


def _tc_final(Ps, Pv, ws):
    bn = 1000
    grid = (_NQ // bn,)
    row2 = lambda shape: pl.BlockSpec(shape, lambda i: (0, i, 0))
    full = lambda a: pl.BlockSpec(a.shape, lambda i: (0,) * a.ndim)
    return pl.pallas_call(
        _final_body,
        grid=grid,
        in_specs=[row2((2, bn, 64)), row2((2, bn, 48))] + [full(a) for a in ws],
        out_specs=pl.BlockSpec((bn, 16), lambda i: (i, 0)),
        out_shape=jax.ShapeDtypeStruct((_NQ, 16), jnp.float32),
    )(Ps, Pv, *ws)


def kernel(pos_query, pos_compose, node_attr_compose_sca, node_attr_compose_vec,
           params, edge_index_q_cps_knn):
    f32 = jnp.float32
    src = edge_index_q_cps_knn[0].astype(jnp.int32)
    dst = edge_index_q_cps_knn[1].astype(jnp.int32)
    nvf = node_attr_compose_vec.reshape(_NCTX, -1).astype(f32)
    pc8 = jnp.pad(pos_compose.astype(f32), ((0, 0), (0, 5)))
    pq8 = jnp.pad(pos_query.astype(f32), ((0, 0), (0, 5)))

    mp = params['msg']
    r1 = lambda b: b.astype(f32).reshape(1, -1)

    ng = mp['node_gv']
    ws1 = [_kron3(ng['lin_vector_W']), jnp.asarray(_red3(32)),
           ng['lin_scalar_W'][:, :32].T, ng['lin_scalar_W'][:, 32:].T,
           _kron3(ng['lin_vector2_W']), ng['gates_W'].T, r1(ng['gates_b']),
           jnp.asarray(_exp3(16))]

    eg = mp['edge_gvp']
    og = mp['out_gv']
    offs = jnp.linspace(0., _CUT, _EC).reshape(1, _EC)
    w = params['vec_exp_W'][:, 0]
    Kevf = jnp.zeros((8, 48), f32).at[:3, :].set(
        jnp.kron(w[None, :], jnp.eye(3, dtype=f32)).reshape(3, 48))
    ws3 = [offs, Kevf,
           _kron3(eg['lin_vector_W']), jnp.asarray(_red3(16)),
           eg['lin_scalar_W'][:, :16].T, eg['lin_scalar_W'][:, 16:].T,
           _kron3(eg['lin_vector2_W']), eg['gates_W'].T, r1(eg['gates_b']),
           jnp.asarray(_exp3(16)), _kron3(eg['act_vec_W']),
           mp['sca_W'].T, r1(mp['sca_b']),
           mp['e2n_W'].T, r1(mp['e2n_b']),
           mp['n2e_W'].T, r1(mp['n2e_b']),
           _kron3(mp['edge_vn_W']), r1(jnp.repeat(mp['edge_vn_b'], 3)),
           _kron3(og['lin_vector_W']), jnp.asarray(_red3(16)),
           og['lin_scalar_W'][:, :16].T, og['lin_scalar_W'][:, 16:].T,
           _kron3(og['lin_vector2_W']), og['gates_W'].T, r1(og['gates_b']),
           jnp.asarray(_exp3(16))]

    cg = params['cls_gvp']
    gg = params['cls_gv']
    Wsng = jnp.zeros((16, 16), f32).at[:, :13].set(gg['lin_scalar_W'][:, :16].T)
    Wssg = jnp.zeros((64, 16), f32).at[:, :13].set(gg['lin_scalar_W'][:, 16:].T)
    ws5 = [_kron3(cg['lin_vector_W']), jnp.asarray(_red3(16)),
           cg['lin_scalar_W'][:, :16].T, cg['lin_scalar_W'][:, 16:].T,
           _kron3(cg['lin_vector2_W']), cg['gates_W'].T, r1(cg['gates_b']),
           jnp.asarray(_exp3(16)), _kron3(cg['act_vec_W']),
           _kron3(gg['lin_vector_W']), jnp.asarray(_red3(16)),
           Wsng, Wssg]

    T = _tc_node(node_attr_compose_sca.astype(f32), nvf, pc8, ws1)
    G, Gq = _sc_gather(T, pq8, dst, src)
    Os, Ov = _tc_edge(G, Gq, ws3)
    zs = jnp.zeros((_RPT, 64), f32)
    zv = jnp.zeros((_RPT, 48), f32)
    Ps, Pv = _sc_scatter(Os, Ov, src, zs, zv)
    y16 = _tc_final(Ps[:, :_NQ], Pv[:, :_NQ], ws5)
    return y16[:, :13]
